# baseline (speedup 1.0000x reference)
import functools

import jax
import jax.numpy as jnp
from jax import lax
from jax.experimental import pallas as pl
from jax.experimental.pallas import tpu as pltpu
from jax.experimental.pallas import tpu_sc as plsc

D_MODEL = 64
SCALE = 8.0

NUM_CORES = 2
NUM_SUBCORES = 16
NUM_WORKERS = NUM_CORES * NUM_SUBCORES

LANE = 128
GROUP = 8
NBUF = 4


def _emb_body(xt_hbm, tp_hbm, out_hbm,
              raw, rows, tr, gsem, ssem, *, seq, na):
    wid = lax.axis_index("s") * NUM_CORES + lax.axis_index("c")
    a0 = wid * LANE

    iota = lax.iota(jnp.int32, 16)
    row_vs = [iota + (a8 * 16) for a8 in range(LANE // 16)]

    def stage_group(g):
        pltpu.sync_copy(xt_hbm.at[pl.ds(g * GROUP, GROUP), pl.ds(a0, LANE)],
                        raw.at[lax.rem(g, 2)])

    def fire(i, b):
        pltpu.async_copy(
            tp_hbm.at[raw.at[lax.rem(b // GROUP, 2), lax.rem(b, GROUP)]],
            rows.at[i], gsem)

    def wait_gather(i):
        pltpu.make_async_copy(tp_hbm.at[raw.at[0, 0]], rows.at[i],
                              gsem).wait()

    def transpose(i, j):
        @plsc.parallel_loop(0, LANE, unroll=2)
        def _(a):
            col_v = jnp.broadcast_to(a, (16,))
            for g in range(D_MODEL // 16):
                val = rows[i, a, pl.ds(g * 16, 16)]
                plsc.store_scatter(tr.at[j], [row_vs[g], col_v], val * SCALE)

    def store(j, b):
        pltpu.async_copy(tr.at[j, slice(None), pl.ds(0, LANE)],
                         out_hbm.at[b, slice(None), pl.ds(a0, LANE)],
                         ssem)

    def wait_store(j):
        pltpu.make_async_copy(tr.at[j, slice(None), pl.ds(0, LANE)],
                              out_hbm.at[0, slice(None), pl.ds(a0, LANE)],
                              ssem).wait()

    stage_group(0)
    for i in range(NBUF):
        fire(i, i)

    def step(b, carry):
        i = lax.rem(b, NBUF)
        j = lax.rem(b, 2)

        @pl.when((lax.rem(b, GROUP) == NBUF) & (b < seq - NBUF))
        def _():
            stage_group((b + NBUF) // GROUP)

        @pl.when(b >= 2)
        def _():
            wait_store(j)

        wait_gather(i)
        transpose(i, j)
        store(j, b)

        @pl.when(b < seq - NBUF)
        def _():
            fire(i, b + NBUF)

        return carry

    lax.fori_loop(0, seq, step, 0)
    wait_store(0)
    wait_store(1)


def _fmt_body(t_ref, o_ref):
    eye = jnp.eye(D_MODEL, dtype=jnp.float32)
    t = t_ref[...]
    o_ref[:, 0:D_MODEL] = jax.lax.dot_general(
        t, eye, (((0,), (0,)), ((), ())),
        preferred_element_type=jnp.float32)


@jax.jit
def _format_table(tin):
    d, v = tin.shape
    cb = 512
    grid = (v + cb - 1) // cb
    return pl.pallas_call(
        _fmt_body,
        grid=(grid,),
        in_specs=[pl.BlockSpec((d, cb), lambda j: (0, j))],
        out_specs=pl.BlockSpec((cb, 2 * D_MODEL), lambda j: (j, 0)),
        out_shape=jax.ShapeDtypeStruct((v, 2 * D_MODEL), jnp.float32),
    )(tin)


@jax.jit
def _emb(xt, tp):
    seq, na = xt.shape
    mesh = plsc.VectorSubcoreMesh(core_axis_name="c", subcore_axis_name="s")
    kern = pl.kernel(
        functools.partial(_emb_body, seq=seq, na=na),
        out_type=jax.ShapeDtypeStruct((seq, D_MODEL, na), jnp.float32),
        mesh=mesh,
        scratch_types=[
            pltpu.VMEM((2, GROUP, LANE), jnp.int32),
            pltpu.VMEM((NBUF, LANE, 2 * D_MODEL), jnp.float32),
            pltpu.VMEM((2, D_MODEL, LANE + 1), jnp.float32),
            pltpu.SemaphoreType.DMA,
            pltpu.SemaphoreType.DMA,
        ],
        compiler_params=pltpu.CompilerParams(use_tc_tiling_on_sc=True,
                                             needs_layout_passes=False),
    )
    return kern(xt, tp)


def kernel(x, table):
    na, seq = x.shape
    assert na == NUM_WORKERS * LANE and seq % GROUP == 0 and seq % NBUF == 0
    xt = jnp.transpose(x.astype(jnp.int32))
    tp = _format_table(jnp.transpose(table))
    out_t = _emb(xt, tp)
    return jnp.transpose(out_t, (2, 0, 1))


# speedup vs baseline: 2.0105x; 2.0105x over previous
import functools

import jax
import jax.numpy as jnp
from jax import lax
from jax.experimental import pallas as pl
from jax.experimental.pallas import tpu as pltpu
from jax.experimental.pallas import tpu_sc as plsc

D_MODEL = 64
SCALE = 8.0

NUM_CORES = 2
NUM_SUBCORES = 16
NUM_WORKERS = NUM_CORES * NUM_SUBCORES

LANE = 128
GROUP = 8
NBUF = 4


def _emb_body(xt_hbm, tp_hbm, out_hbm,
              raw, rows, tr, gsem, ssem, *, seq, na):
    wid = lax.axis_index("s") * NUM_CORES + lax.axis_index("c")
    a0 = wid * LANE

    iota = lax.iota(jnp.int32, 16)
    row_vs = [iota + (a8 * 16) for a8 in range(LANE // 16)]

    def stage_group(g):
        pltpu.sync_copy(xt_hbm.at[pl.ds(g * GROUP, GROUP), pl.ds(a0, LANE)],
                        raw.at[lax.rem(g, 2)])

    def fire(i, b):
        pltpu.async_copy(
            tp_hbm.at[raw.at[lax.rem(b // GROUP, 2), lax.rem(b, GROUP)]],
            rows.at[i], gsem)

    def wait_gather(i):
        pltpu.make_async_copy(tp_hbm.at[raw.at[0, 0]], rows.at[i],
                              gsem).wait()

    def transpose(i, j):
        @plsc.parallel_loop(0, LANE, unroll=2)
        def _(a):
            col_v = jnp.broadcast_to(a, (16,))
            for g in range(D_MODEL // 16):
                val = rows[i, a, pl.ds(g * 16, 16)]
                plsc.store_scatter(tr.at[j], [row_vs[g], col_v], val * SCALE)

    def store(j, b):
        pltpu.async_copy(tr.at[j, slice(None), pl.ds(0, LANE)],
                         out_hbm.at[b, slice(None), pl.ds(a0, LANE)],
                         ssem)

    def wait_store(j):
        pltpu.make_async_copy(tr.at[j, slice(None), pl.ds(0, LANE)],
                              out_hbm.at[0, slice(None), pl.ds(a0, LANE)],
                              ssem).wait()

    stage_group(0)
    for i in range(NBUF):
        fire(i, i)

    def step(b, carry):
        i = lax.rem(b, NBUF)
        j = lax.rem(b, 2)

        @pl.when((lax.rem(b, GROUP) == NBUF) & (b < seq - NBUF))
        def _():
            stage_group((b + NBUF) // GROUP)

        @pl.when(b >= 2)
        def _():
            wait_store(j)

        wait_gather(i)
        transpose(i, j)
        store(j, b)

        @pl.when(b < seq - NBUF)
        def _():
            fire(i, b + NBUF)

        return carry

    lax.fori_loop(0, seq, step, 0)
    wait_store(0)
    wait_store(1)


def _fmt_body(t_ref, o_ref):
    o_ref[:, 0:D_MODEL] = jnp.transpose(t_ref[...], (1, 0))


@jax.jit
def _format_table(tin):
    d, v = tin.shape
    cb = 8192
    grid = (v + cb - 1) // cb
    return pl.pallas_call(
        _fmt_body,
        grid=(grid,),
        in_specs=[pl.BlockSpec((d, cb), lambda j: (0, j))],
        out_specs=pl.BlockSpec((cb, 2 * D_MODEL), lambda j: (j, 0)),
        out_shape=jax.ShapeDtypeStruct((v, 2 * D_MODEL), jnp.float32),
    )(tin)


@jax.jit
def _emb(xt, tp):
    seq, na = xt.shape
    mesh = plsc.VectorSubcoreMesh(core_axis_name="c", subcore_axis_name="s")
    kern = pl.kernel(
        functools.partial(_emb_body, seq=seq, na=na),
        out_type=jax.ShapeDtypeStruct((seq, D_MODEL, na), jnp.float32),
        mesh=mesh,
        scratch_types=[
            pltpu.VMEM((2, GROUP, LANE), jnp.int32),
            pltpu.VMEM((NBUF, LANE, 2 * D_MODEL), jnp.float32),
            pltpu.VMEM((2, D_MODEL, LANE + 1), jnp.float32),
            pltpu.SemaphoreType.DMA,
            pltpu.SemaphoreType.DMA,
        ],
        compiler_params=pltpu.CompilerParams(use_tc_tiling_on_sc=True,
                                             needs_layout_passes=False),
    )
    return kern(xt, tp)


def kernel(x, table):
    na, seq = x.shape
    assert na == NUM_WORKERS * LANE and seq % GROUP == 0 and seq % NBUF == 0
    xt = jnp.transpose(x.astype(jnp.int32))
    tp = _format_table(jnp.transpose(table))
    out_t = _emb(xt, tp)
    return jnp.transpose(out_t, (2, 0, 1))


# speedup vs baseline: 2.0155x; 1.0025x over previous
import functools

import jax
import jax.numpy as jnp
from jax import lax
from jax.experimental import pallas as pl
from jax.experimental.pallas import tpu as pltpu
from jax.experimental.pallas import tpu_sc as plsc

D_MODEL = 64
SCALE = 8.0

NUM_CORES = 2
NUM_SUBCORES = 16
NUM_WORKERS = NUM_CORES * NUM_SUBCORES

LANE = 128
GROUP = 8
NBUF = 4


def _emb_body(xt_hbm, tp_hbm, out_hbm,
              raw, rows, tr, gsem, ssem, *, seq, na):
    wid = lax.axis_index("s") * NUM_CORES + lax.axis_index("c")
    a0 = wid * LANE

    iota = lax.iota(jnp.int32, 16)
    row_vs = [iota + (a8 * 16) for a8 in range(LANE // 16)]

    def stage_group(g):
        pltpu.sync_copy(xt_hbm.at[pl.ds(g * GROUP, GROUP), pl.ds(a0, LANE)],
                        raw.at[lax.rem(g, 2)])

    def fire(i, b):
        pltpu.async_copy(
            tp_hbm.at[raw.at[lax.rem(b // GROUP, 2), lax.rem(b, GROUP)]],
            rows.at[i], gsem)

    def wait_gather(i):
        pltpu.make_async_copy(tp_hbm.at[raw.at[0, 0]], rows.at[i],
                              gsem).wait()

    def transpose(i, trj):
        @plsc.parallel_loop(0, LANE, unroll=4, carry=jnp.zeros((16,), jnp.int32))
        def _(a, col_v):
            for g in range(D_MODEL // 16):
                val = rows[i, a, pl.ds(g * 16, 16)]
                plsc.store_scatter(trj, [row_vs[g], col_v], val * SCALE)
            return col_v + 1

    def store(trj, b):
        pltpu.async_copy(trj.at[slice(None), pl.ds(0, LANE)],
                         out_hbm.at[b, slice(None), pl.ds(a0, LANE)],
                         ssem)

    def wait_store(trj):
        pltpu.make_async_copy(trj.at[slice(None), pl.ds(0, LANE)],
                              out_hbm.at[0, slice(None), pl.ds(a0, LANE)],
                              ssem).wait()

    stage_group(0)
    for i in range(NBUF):
        fire(i, i)

    def step(p, carry):
        b0 = 2 * p
        for j in range(2):
            b = b0 + j
            trj = tr.at[j]

            @pl.when((lax.rem(b, GROUP) == NBUF) & (b < seq - NBUF))
            def _():
                stage_group((b + NBUF) // GROUP)

            @pl.when(b >= 2)
            def _():
                wait_store(trj)

            wait_gather(lax.rem(b, NBUF))
            transpose(lax.rem(b, NBUF), trj)
            store(trj, b)

            @pl.when(b < seq - NBUF)
            def _():
                fire(lax.rem(b, NBUF), b + NBUF)

        return carry

    lax.fori_loop(0, seq // 2, step, 0)
    wait_store(tr.at[0])
    wait_store(tr.at[1])


def _fmt_body(t_ref, o_ref):
    o_ref[:, 0:D_MODEL] = jnp.transpose(t_ref[...], (1, 0))


@jax.jit
def _format_table(tin):
    d, v = tin.shape
    cb = 8192
    grid = (v + cb - 1) // cb
    return pl.pallas_call(
        _fmt_body,
        grid=(grid,),
        in_specs=[pl.BlockSpec((d, cb), lambda j: (0, j))],
        out_specs=pl.BlockSpec((cb, 2 * D_MODEL), lambda j: (j, 0)),
        out_shape=jax.ShapeDtypeStruct((v, 2 * D_MODEL), jnp.float32),
    )(tin)


@jax.jit
def _emb(xt, tp):
    seq, na = xt.shape
    mesh = plsc.VectorSubcoreMesh(core_axis_name="c", subcore_axis_name="s")
    kern = pl.kernel(
        functools.partial(_emb_body, seq=seq, na=na),
        out_type=jax.ShapeDtypeStruct((seq, D_MODEL, na), jnp.float32),
        mesh=mesh,
        scratch_types=[
            pltpu.VMEM((2, GROUP, LANE), jnp.int32),
            pltpu.VMEM((NBUF, LANE, 2 * D_MODEL), jnp.float32),
            pltpu.VMEM((2, D_MODEL, LANE + 1), jnp.float32),
            pltpu.SemaphoreType.DMA,
            pltpu.SemaphoreType.DMA,
        ],
        compiler_params=pltpu.CompilerParams(use_tc_tiling_on_sc=True,
                                             needs_layout_passes=False),
    )
    return kern(xt, tp)


def kernel(x, table):
    na, seq = x.shape
    assert na == NUM_WORKERS * LANE and seq % GROUP == 0 and seq % NBUF == 0
    xt = jnp.transpose(x.astype(jnp.int32))
    tp = _format_table(jnp.transpose(table))
    out_t = _emb(xt, tp)
    return jnp.transpose(out_t, (2, 0, 1))


# speedup vs baseline: 2.0635x; 1.0238x over previous
import functools

import jax
import jax.numpy as jnp
from jax import lax
from jax.experimental import pallas as pl
from jax.experimental.pallas import tpu as pltpu
from jax.experimental.pallas import tpu_sc as plsc

D_MODEL = 64
SCALE = 8.0

NUM_CORES = 2
NUM_SUBCORES = 16
NUM_WORKERS = NUM_CORES * NUM_SUBCORES

LANE = 128
GROUP = 8
NBUF = 4


def _emb_body(xt_hbm, tp_hbm, out_hbm,
              raw, rows, tr, gsem, ssem, *, seq, na):
    wid = lax.axis_index("s") * NUM_CORES + lax.axis_index("c")
    a0 = wid * LANE

    iota = lax.iota(jnp.int32, 16)
    row_vs = [iota + (a8 * 16) for a8 in range(LANE // 16)]

    def stage_group(g):
        pltpu.sync_copy(xt_hbm.at[pl.ds(g * GROUP, GROUP), pl.ds(a0, LANE)],
                        raw.at[lax.rem(g, 2)])

    def fire(i, b):
        pltpu.async_copy(
            tp_hbm.at[raw.at[lax.rem(b // GROUP, 2), lax.rem(b, GROUP)]],
            rows.at[i], gsem)

    def wait_gather(i):
        pltpu.make_async_copy(tp_hbm.at[raw.at[0, 0]], rows.at[i],
                              gsem).wait()

    def transpose(i, trj):
        @plsc.parallel_loop(0, LANE, unroll=4)
        def _(a):
            col_v = jnp.broadcast_to(a, (16,))
            for g in range(D_MODEL // 32):
                val = rows[i, a, pl.ds(g * 16, 16)]
                plsc.store_scatter(trj, [row_vs[g], col_v], val * SCALE)

        @plsc.parallel_loop(D_MODEL // 2, D_MODEL, unroll=4)
        def _(d):
            col_v = jnp.broadcast_to(d, (16,))
            for a8 in range(LANE // 16):
                val = plsc.load_gather(rows.at[i], [row_vs[a8], col_v])
                trj[d, pl.ds(a8 * 16, 16)] = val * SCALE

    def store(trj, b):
        pltpu.async_copy(trj.at[slice(None), pl.ds(0, LANE)],
                         out_hbm.at[b, slice(None), pl.ds(a0, LANE)],
                         ssem)

    def wait_store(trj):
        pltpu.make_async_copy(trj.at[slice(None), pl.ds(0, LANE)],
                              out_hbm.at[0, slice(None), pl.ds(a0, LANE)],
                              ssem).wait()

    stage_group(0)
    for i in range(NBUF):
        fire(i, i)

    def step(p, carry):
        b0 = 2 * p
        for j in range(2):
            b = b0 + j
            trj = tr.at[j]

            @pl.when((lax.rem(b, GROUP) == NBUF) & (b < seq - NBUF))
            def _():
                stage_group((b + NBUF) // GROUP)

            @pl.when(b >= 2)
            def _():
                wait_store(trj)

            wait_gather(lax.rem(b, NBUF))
            transpose(lax.rem(b, NBUF), trj)
            store(trj, b)

            @pl.when(b < seq - NBUF)
            def _():
                fire(lax.rem(b, NBUF), b + NBUF)

        return carry

    lax.fori_loop(0, seq // 2, step, 0)
    wait_store(tr.at[0])
    wait_store(tr.at[1])


def _fmt_body(t_ref, o_ref):
    o_ref[:, 0:D_MODEL] = jnp.transpose(t_ref[...], (1, 0))


@jax.jit
def _format_table(tin):
    d, v = tin.shape
    cb = 8192
    grid = (v + cb - 1) // cb
    return pl.pallas_call(
        _fmt_body,
        grid=(grid,),
        in_specs=[pl.BlockSpec((d, cb), lambda j: (0, j))],
        out_specs=pl.BlockSpec((cb, 2 * D_MODEL), lambda j: (j, 0)),
        out_shape=jax.ShapeDtypeStruct((v, 2 * D_MODEL), jnp.float32),
    )(tin)


@jax.jit
def _emb(xt, tp):
    seq, na = xt.shape
    mesh = plsc.VectorSubcoreMesh(core_axis_name="c", subcore_axis_name="s")
    kern = pl.kernel(
        functools.partial(_emb_body, seq=seq, na=na),
        out_type=jax.ShapeDtypeStruct((seq, D_MODEL, na), jnp.float32),
        mesh=mesh,
        scratch_types=[
            pltpu.VMEM((2, GROUP, LANE), jnp.int32),
            pltpu.VMEM((NBUF, LANE, 2 * D_MODEL), jnp.float32),
            pltpu.VMEM((2, D_MODEL, LANE + 1), jnp.float32),
            pltpu.SemaphoreType.DMA,
            pltpu.SemaphoreType.DMA,
        ],
        compiler_params=pltpu.CompilerParams(use_tc_tiling_on_sc=True,
                                             needs_layout_passes=False),
    )
    return kern(xt, tp)


def kernel(x, table):
    na, seq = x.shape
    assert na == NUM_WORKERS * LANE and seq % GROUP == 0 and seq % NBUF == 0
    xt = jnp.transpose(x.astype(jnp.int32))
    tp = _format_table(jnp.transpose(table))
    out_t = _emb(xt, tp)
    return jnp.transpose(out_t, (2, 0, 1))


# speedup vs baseline: 2.1018x; 1.0186x over previous
import functools

import jax
import jax.numpy as jnp
from jax import lax
from jax.experimental import pallas as pl
from jax.experimental.pallas import tpu as pltpu
from jax.experimental.pallas import tpu_sc as plsc

D_MODEL = 64
SCALE = 8.0

NUM_CORES = 2
NUM_SUBCORES = 16
NUM_WORKERS = NUM_CORES * NUM_SUBCORES

LANE = 128
GROUP = 8
NBUF = 4


def _emb_body(xt_hbm, tp_hbm, out_hbm,
              raw, rows, tr, gsem, ssem, *, seq, na):
    wid = lax.axis_index("s") * NUM_CORES + lax.axis_index("c")
    a0 = wid * LANE

    iota = lax.iota(jnp.int32, 16)
    row_vs = [iota + (a8 * 16) for a8 in range(LANE // 16)]

    def stage_group(g):
        pltpu.sync_copy(xt_hbm.at[pl.ds(g * GROUP, GROUP), pl.ds(a0, LANE)],
                        raw.at[lax.rem(g, 2)])

    def fire(i, b):
        pltpu.async_copy(
            tp_hbm.at[raw.at[lax.rem(b // GROUP, 2), lax.rem(b, GROUP)]],
            rows.at[i], gsem)

    def wait_gather(i):
        pltpu.make_async_copy(tp_hbm.at[raw.at[0, 0]], rows.at[i],
                              gsem).wait()

    def transpose(i, trj):
        @plsc.parallel_loop(0, LANE, unroll=4)
        def _(a):
            col_v = jnp.broadcast_to(a, (16,))
            for g in range(D_MODEL // 32):
                val = rows[i, a, pl.ds(g * 16, 16)]
                plsc.store_scatter(trj, [row_vs[g], col_v], val * SCALE)

        @plsc.parallel_loop(D_MODEL // 2, D_MODEL, unroll=4)
        def _(d):
            col_v = jnp.broadcast_to(d, (16,))
            for a8 in range(LANE // 16):
                val = plsc.load_gather(rows.at[i], [row_vs[a8], col_v])
                trj[d, pl.ds(a8 * 16, 16)] = val * SCALE

    def store(trj, b):
        pltpu.async_copy(trj.at[slice(None), pl.ds(0, LANE)],
                         out_hbm.at[b, slice(None), pl.ds(a0, LANE)],
                         ssem)

    def wait_store(trj):
        pltpu.make_async_copy(trj.at[slice(None), pl.ds(0, LANE)],
                              out_hbm.at[0, slice(None), pl.ds(a0, LANE)],
                              ssem).wait()

    stage_group(0)
    for i in range(NBUF):
        fire(i, i)

    def step(p, carry):
        b0 = 2 * p
        for j in range(2):
            b = b0 + j
            trj = tr.at[j]

            @pl.when((lax.rem(b, GROUP) == NBUF) & (b < seq - NBUF))
            def _():
                stage_group((b + NBUF) // GROUP)

            @pl.when(b >= 2)
            def _():
                wait_store(trj)

            wait_gather(lax.rem(b, NBUF))
            transpose(lax.rem(b, NBUF), trj)
            store(trj, b)

            @pl.when(b < seq - NBUF)
            def _():
                fire(lax.rem(b, NBUF), b + NBUF)

        return carry

    lax.fori_loop(0, seq // 2, step, 0)
    wait_store(tr.at[0])
    wait_store(tr.at[1])


def _fmt_body(t_ref, o_ref):
    o_ref[:, 0:D_MODEL] = jnp.transpose(t_ref[...], (1, 0))


@jax.jit
def _format_table(tin):
    d, v = tin.shape
    cb = 16384
    grid = (v + cb - 1) // cb
    return pl.pallas_call(
        _fmt_body,
        grid=(grid,),
        in_specs=[pl.BlockSpec((d, cb), lambda j: (0, j))],
        out_specs=pl.BlockSpec((cb, 2 * D_MODEL), lambda j: (j, 0)),
        out_shape=jax.ShapeDtypeStruct((v, 2 * D_MODEL), jnp.float32),
    )(tin)


@jax.jit
def _emb(xt, tp):
    seq, na = xt.shape
    mesh = plsc.VectorSubcoreMesh(core_axis_name="c", subcore_axis_name="s")
    kern = pl.kernel(
        functools.partial(_emb_body, seq=seq, na=na),
        out_type=jax.ShapeDtypeStruct((seq, D_MODEL, na), jnp.float32),
        mesh=mesh,
        scratch_types=[
            pltpu.VMEM((2, GROUP, LANE), jnp.int32),
            pltpu.VMEM((NBUF, LANE, 2 * D_MODEL), jnp.float32),
            pltpu.VMEM((2, D_MODEL, LANE + 1), jnp.float32),
            pltpu.SemaphoreType.DMA,
            pltpu.SemaphoreType.DMA,
        ],
        compiler_params=pltpu.CompilerParams(use_tc_tiling_on_sc=True,
                                             needs_layout_passes=False),
    )
    return kern(xt, tp)


def kernel(x, table):
    na, seq = x.shape
    assert na == NUM_WORKERS * LANE and seq % GROUP == 0 and seq % NBUF == 0
    xt = jnp.transpose(x.astype(jnp.int32))
    tp = _format_table(jnp.transpose(table))
    out_t = _emb(xt, tp)
    return jnp.transpose(out_t, (2, 0, 1))


# speedup vs baseline: 2.1061x; 1.0020x over previous
import functools

import jax
import jax.numpy as jnp
from jax import lax
from jax.experimental import pallas as pl
from jax.experimental.pallas import tpu as pltpu
from jax.experimental.pallas import tpu_sc as plsc

D_MODEL = 64
SCALE = 8.0

NUM_CORES = 2
NUM_SUBCORES = 16
NUM_WORKERS = NUM_CORES * NUM_SUBCORES

LANE = 128
GROUP = 8
NBUF = 4


def _emb_body(xt_hbm, tp_hbm, out_hbm,
              raw, rows, tr, gsem, ssem, *, seq, na):
    wid = lax.axis_index("s") * NUM_CORES + lax.axis_index("c")
    a0 = wid * LANE

    iota = lax.iota(jnp.int32, 16)
    row_vs = [iota + (a8 * 16) for a8 in range(LANE // 16)]

    def stage_group(g):
        pltpu.sync_copy(xt_hbm.at[pl.ds(g * GROUP, GROUP), pl.ds(a0, LANE)],
                        raw.at[lax.rem(g, 2)])

    def fire(i, b):
        pltpu.async_copy(
            tp_hbm.at[raw.at[lax.rem(b // GROUP, 2), lax.rem(b, GROUP)]],
            rows.at[i], gsem)

    def wait_gather(i):
        pltpu.make_async_copy(tp_hbm.at[raw.at[0, 0]], rows.at[i],
                              gsem).wait()

    def transpose(i, trj):
        @plsc.parallel_loop(0, LANE, unroll=8)
        def _(a):
            col_v = jnp.broadcast_to(a, (16,))
            for g in range(D_MODEL // 32):
                val = rows[i, a, pl.ds(g * 16, 16)]
                plsc.store_scatter(trj, [row_vs[g], col_v], val * SCALE)

        @plsc.parallel_loop(D_MODEL // 2, D_MODEL, unroll=8)
        def _(d):
            col_v = jnp.broadcast_to(d, (16,))
            for a8 in range(LANE // 16):
                val = plsc.load_gather(rows.at[i], [row_vs[a8], col_v])
                trj[d, pl.ds(a8 * 16, 16)] = val * SCALE

    def store(trj, b):
        pltpu.async_copy(trj.at[slice(None), pl.ds(0, LANE)],
                         out_hbm.at[b, slice(None), pl.ds(a0, LANE)],
                         ssem)

    def wait_store(trj):
        pltpu.make_async_copy(trj.at[slice(None), pl.ds(0, LANE)],
                              out_hbm.at[0, slice(None), pl.ds(a0, LANE)],
                              ssem).wait()

    stage_group(0)
    for i in range(NBUF):
        fire(i, i)

    def step(p, carry):
        b0 = 2 * p
        for j in range(2):
            b = b0 + j
            trj = tr.at[j]

            @pl.when((lax.rem(b, GROUP) == NBUF) & (b < seq - NBUF))
            def _():
                stage_group((b + NBUF) // GROUP)

            @pl.when(b >= 2)
            def _():
                wait_store(trj)

            wait_gather(lax.rem(b, NBUF))
            transpose(lax.rem(b, NBUF), trj)
            store(trj, b)

            @pl.when(b < seq - NBUF)
            def _():
                fire(lax.rem(b, NBUF), b + NBUF)

        return carry

    lax.fori_loop(0, seq // 2, step, 0)
    wait_store(tr.at[0])
    wait_store(tr.at[1])


def _fmt_body(t_ref, o_ref):
    o_ref[:, 0:D_MODEL] = jnp.transpose(t_ref[...], (1, 0))


@jax.jit
def _format_table(tin):
    d, v = tin.shape
    cb = 16384
    grid = (v + cb - 1) // cb
    return pl.pallas_call(
        _fmt_body,
        grid=(grid,),
        in_specs=[pl.BlockSpec((d, cb), lambda j: (0, j))],
        out_specs=pl.BlockSpec((cb, 2 * D_MODEL), lambda j: (j, 0)),
        out_shape=jax.ShapeDtypeStruct((v, 2 * D_MODEL), jnp.float32),
    )(tin)


@jax.jit
def _emb(xt, tp):
    seq, na = xt.shape
    mesh = plsc.VectorSubcoreMesh(core_axis_name="c", subcore_axis_name="s")
    kern = pl.kernel(
        functools.partial(_emb_body, seq=seq, na=na),
        out_type=jax.ShapeDtypeStruct((seq, D_MODEL, na), jnp.float32),
        mesh=mesh,
        scratch_types=[
            pltpu.VMEM((2, GROUP, LANE), jnp.int32),
            pltpu.VMEM((NBUF, LANE, 2 * D_MODEL), jnp.float32),
            pltpu.VMEM((2, D_MODEL, LANE + 1), jnp.float32),
            pltpu.SemaphoreType.DMA,
            pltpu.SemaphoreType.DMA,
        ],
        compiler_params=pltpu.CompilerParams(use_tc_tiling_on_sc=True,
                                             needs_layout_passes=False),
    )
    return kern(xt, tp)


def kernel(x, table):
    na, seq = x.shape
    assert na == NUM_WORKERS * LANE and seq % GROUP == 0 and seq % NBUF == 0
    xt = jnp.transpose(x.astype(jnp.int32))
    tp = _format_table(jnp.transpose(table))
    out_t = _emb(xt, tp)
    return jnp.transpose(out_t, (2, 0, 1))
